# baseline (speedup 1.0000x reference)
import functools

import jax
import jax.numpy as jnp
from jax import lax
from jax.experimental import pallas as pl
from jax.experimental.pallas import tpu as pltpu
from jax.experimental.pallas import tpu_sc as plsc

_N = 100000
_E = 1600000
_H = 32
_MSG = 2

_NC = 2
_NS = 16
_NW = _NC * _NS
_EW = _E // _NW
_SG = 125
_NSG = 8
_CH = _SG * _NSG
_CPW = _EW // _CH
_ROWS = _E // _SG


def _node_tables_body(x_ref, wa_ref, wb_ref, ca_ref, cb_ref, a_ref, b_ref):
    xb = x_ref[...]
    a_ref[...] = (
        jnp.dot(xb, wa_ref[...], preferred_element_type=jnp.float32) + ca_ref[...]
    )
    b_ref[...] = (
        jnp.dot(xb, wb_ref[...], preferred_element_type=jnp.float32) + cb_ref[...]
    )


def _node_tables(x, wa_t, wb_t, ca, cb):
    bn = 10000
    grid = (_N // bn,)
    return pl.pallas_call(
        _node_tables_body,
        grid=grid,
        in_specs=[
            pl.BlockSpec((bn, 2), lambda i: (i, 0)),
            pl.BlockSpec((2, _H), lambda i: (0, 0)),
            pl.BlockSpec((2, _H), lambda i: (0, 0)),
            pl.BlockSpec((1, _H), lambda i: (0, 0)),
            pl.BlockSpec((1, _H), lambda i: (0, 0)),
        ],
        out_specs=[
            pl.BlockSpec((bn, _H), lambda i: (i, 0)),
            pl.BlockSpec((bn, _H), lambda i: (i, 0)),
        ],
        out_shape=[
            jax.ShapeDtypeStruct((_N, _H), jnp.float32),
            jax.ShapeDtypeStruct((_N, _H), jnp.float32),
        ],
    )(x, wa_t, wb_t, ca, cb)


def _gather_body(a_hbm, b_hbm, dst_hbm, src_hbm, g_hbm, dstv, srcv, rows, sem_g):
    c = lax.axis_index("c")
    s = lax.axis_index("s")
    wid = s * _NC + c
    base_row = wid * (_EW // _SG)

    def chunk_body(k, carry):
        row0 = base_row + k * _NSG
        pltpu.sync_copy(dst_hbm.at[pl.ds(row0, _NSG)], dstv)
        pltpu.sync_copy(src_hbm.at[pl.ds(row0, _NSG)], srcv)

        def fire_a(j, cy):
            pltpu.async_copy(a_hbm.at[dstv.at[j]], rows.at[j], sem_g)
            return cy

        lax.fori_loop(0, _NSG, fire_a, 0)

        def drain_a(j, cy):
            pltpu.make_async_copy(a_hbm.at[dstv.at[j]], rows.at[j], sem_g).wait()
            return cy

        lax.fori_loop(0, _NSG, drain_a, 0)

        def fire_b(j, cy):
            pltpu.async_copy(b_hbm.at[srcv.at[j]], rows.at[j], sem_g, add=True)
            return cy

        lax.fori_loop(0, _NSG, fire_b, 0)

        def drain_b(j, cy):
            pltpu.make_async_copy(b_hbm.at[srcv.at[j]], rows.at[j], sem_g).wait()
            return cy

        lax.fori_loop(0, _NSG, drain_b, 0)

        pltpu.sync_copy(rows, g_hbm.at[pl.ds(row0, _NSG)])
        return carry

    lax.fori_loop(0, _CPW, chunk_body, 0)


def _gather_add(a, b, dst3, src3):
    mesh = plsc.VectorSubcoreMesh(
        core_axis_name="c", subcore_axis_name="s", num_cores=_NC, num_subcores=_NS
    )
    fn = pl.kernel(
        _gather_body,
        out_type=jax.ShapeDtypeStruct((_ROWS, _SG, _H), jnp.float32),
        mesh=mesh,
        scratch_types=[
            pltpu.VMEM((_NSG, _SG), jnp.int32),
            pltpu.VMEM((_NSG, _SG), jnp.int32),
            pltpu.VMEM((_NSG, _SG, _H), jnp.float32),
            pltpu.SemaphoreType.DMA,
        ],
        compiler_params=pltpu.CompilerParams(use_tc_tiling_on_sc=False),
    )
    return fn(a, b, dst3, src3)


def _mlp_body(
    gt_ref, yt_ref, ht_ref, wc_ref, wm2_ref, bm2_ref, wdec_ref, bdec_ref,
    hnew_ref, ynew_ref,
):
    yh = jnp.concatenate([yt_ref[...], ht_ref[...]], axis=0)
    m1 = gt_ref[...] + jnp.dot(wc_ref[...], yh, preferred_element_type=jnp.float32)
    m1 = jnp.where(m1 > 0, m1, 0.01 * m1)
    m2 = jnp.dot(wm2_ref[...], m1, preferred_element_type=jnp.float32) + bm2_ref[...]
    m2 = jnp.where(m2 > 0, m2, 0.01 * m2)
    hnew_ref[...] = m2
    lg = jnp.dot(wdec_ref[...], m2, preferred_element_type=jnp.float32) + bdec_ref[...]
    mx = jnp.max(lg, axis=0, keepdims=True)
    ex = jnp.exp(lg - mx)
    ynew_ref[...] = ex / jnp.sum(ex, axis=0, keepdims=True)


def _edge_mlp(gt, yt, ht, wc, wm2, bm2, wdec, bdec):
    be = 12800
    grid = (_E // be,)
    return pl.pallas_call(
        _mlp_body,
        grid=grid,
        in_specs=[
            pl.BlockSpec((_H, be), lambda i: (0, i)),
            pl.BlockSpec((_MSG, be), lambda i: (0, i)),
            pl.BlockSpec((_H, be), lambda i: (0, i)),
            pl.BlockSpec((_H, _MSG + _H), lambda i: (0, 0)),
            pl.BlockSpec((_H, _H), lambda i: (0, 0)),
            pl.BlockSpec((_H, 1), lambda i: (0, 0)),
            pl.BlockSpec((_MSG, _H), lambda i: (0, 0)),
            pl.BlockSpec((_MSG, 1), lambda i: (0, 0)),
        ],
        out_specs=[
            pl.BlockSpec((_H, be), lambda i: (0, i)),
            pl.BlockSpec((_MSG, be), lambda i: (0, i)),
        ],
        out_shape=[
            jax.ShapeDtypeStruct((_H, _E), jnp.float32),
            jax.ShapeDtypeStruct((_MSG, _E), jnp.float32),
        ],
    )(gt, yt, ht, wc, wm2, bm2, wdec, bdec)


def kernel(x, edge_index, h_msg, y_msg, W_in, b_in, W_enc, b_enc, W_m1, b_m1,
           W_m2, b_m2, W_u, b_u, W_dec, b_dec):
    w_m1a = W_m1[:, :_H]
    w_m1b = W_m1[:, _H:2 * _H]
    w_m1c = W_m1[:, 2 * _H:]
    wa = w_m1a @ W_in
    wb = w_m1b @ W_in
    wc = w_m1c @ W_enc
    ca = w_m1a @ b_in + w_m1c @ b_enc + b_m1
    cb = w_m1b @ b_in

    a, b = _node_tables(x, wa.T, wb.T, ca[None, :], cb[None, :])

    dst3 = edge_index[1].reshape(_ROWS, _SG)
    src3 = edge_index[0].reshape(_ROWS, _SG)
    g3 = _gather_add(a, b, dst3, src3)
    gt = g3.reshape(_E, _H).T

    hnewt, ynewt = _edge_mlp(
        gt, y_msg.T, h_msg.T, wc, W_m2, b_m2[:, None], W_dec, b_dec[:, None]
    )
    return hnewt.T, ynewt.T


# speedup vs baseline: 6.0549x; 6.0549x over previous
import functools

import jax
import jax.numpy as jnp
from jax import lax
from jax.experimental import pallas as pl
from jax.experimental.pallas import tpu as pltpu
from jax.experimental.pallas import tpu_sc as plsc

_N = 100000
_E = 1600000
_H = 32
_MSG = 2

_NC = 2
_NS = 16
_NW = _NC * _NS
_EW = _E // _NW
_SG = 125
_NSG = 8
_CH = _SG * _NSG
_CPW = _EW // _CH
_IROWS = _E // _SG
_GQ = _E // 4
_WPK = 8


def _node_tables_body(x_ref, wa_ref, wb_ref, ca_ref, cb_ref, a_ref, b_ref):
    xb = x_ref[...]
    a_ref[...] = (
        jnp.dot(xb, wa_ref[...], preferred_element_type=jnp.float32) + ca_ref[...]
    )
    b_ref[...] = (
        jnp.dot(xb, wb_ref[...], preferred_element_type=jnp.float32) + cb_ref[...]
    )


def _node_tables(x, wa_t, wb_t, ca, cb):
    bn = 10000
    grid = (_N // bn,)
    return pl.pallas_call(
        _node_tables_body,
        grid=grid,
        in_specs=[
            pl.BlockSpec((bn, 2), lambda i: (i, 0)),
            pl.BlockSpec((2, _H), lambda i: (0, 0)),
            pl.BlockSpec((2, _H), lambda i: (0, 0)),
            pl.BlockSpec((1, _H), lambda i: (0, 0)),
            pl.BlockSpec((1, _H), lambda i: (0, 0)),
        ],
        out_specs=[
            pl.BlockSpec((bn, _H), lambda i: (i, 0)),
            pl.BlockSpec((bn, _H), lambda i: (i, 0)),
        ],
        out_shape=[
            jax.ShapeDtypeStruct((_N, _H), jnp.float32),
            jax.ShapeDtypeStruct((_N, _H), jnp.float32),
        ],
    )(x, wa_t, wb_t, ca, cb)


def _gather_body(a_hbm, b_hbm, dst_hbm, src_hbm, g_hbm, dstv, srcv, rows, sem_g):
    c = lax.axis_index("c")
    s = lax.axis_index("s")
    wid = s * _NC + c
    kcol = (wid // _WPK) * _H
    base_irow = wid * (_EW // _SG)
    base_grow = (wid % _WPK) * _EW

    def chunk_body(k, carry):
        irow0 = base_irow + k * _NSG
        grow0 = base_grow + k * _CH
        pltpu.sync_copy(dst_hbm.at[pl.ds(irow0, _NSG)], dstv)
        pltpu.sync_copy(src_hbm.at[pl.ds(irow0, _NSG)], srcv)

        def fire_a(j, cy):
            pltpu.async_copy(a_hbm.at[dstv.at[j]], rows.at[j], sem_g)
            return cy

        lax.fori_loop(0, _NSG, fire_a, 0)

        def drain_a(j, cy):
            pltpu.make_async_copy(a_hbm.at[dstv.at[j]], rows.at[j], sem_g).wait()
            return cy

        lax.fori_loop(0, _NSG, drain_a, 0)

        def fire_b(j, cy):
            pltpu.async_copy(b_hbm.at[srcv.at[j]], rows.at[j], sem_g, add=True)
            return cy

        lax.fori_loop(0, _NSG, fire_b, 0)

        def drain_b(j, cy):
            pltpu.make_async_copy(b_hbm.at[srcv.at[j]], rows.at[j], sem_g).wait()
            return cy

        lax.fori_loop(0, _NSG, drain_b, 0)

        def put_row(j, cy):
            pltpu.sync_copy(
                rows.at[j],
                g_hbm.at[pl.ds(grow0 + j * _SG, _SG), pl.ds(kcol, _H)],
            )
            return cy

        lax.fori_loop(0, _NSG, put_row, 0)
        return carry

    lax.fori_loop(0, _CPW, chunk_body, 0)


def _gather_add(a, b, dst3, src3):
    mesh = plsc.VectorSubcoreMesh(
        core_axis_name="c", subcore_axis_name="s", num_cores=_NC, num_subcores=_NS
    )
    fn = pl.kernel(
        _gather_body,
        out_type=jax.ShapeDtypeStruct((_GQ, 128), jnp.float32),
        mesh=mesh,
        scratch_types=[
            pltpu.VMEM((_NSG, _SG), jnp.int32),
            pltpu.VMEM((_NSG, _SG), jnp.int32),
            pltpu.VMEM((_NSG, _SG, _H), jnp.float32),
            pltpu.SemaphoreType.DMA,
        ],
        compiler_params=pltpu.CompilerParams(use_tc_tiling_on_sc=False),
    )
    return fn(a, b, dst3, src3)


_BQ = 3200
_NBQ = _GQ // _BQ


def _mlp_body(
    g_ref, yt0, yt1, yt2, yt3, ht0, ht1, ht2, ht3,
    wc_ref, wm2_ref, bm2_ref, wdec_ref, bdec_ref,
    hn0, hn1, hn2, hn3, yn0, yn1, yn2, yn3,
):
    gt_full = g_ref[...].T
    yts = (yt0, yt1, yt2, yt3)
    hts = (ht0, ht1, ht2, ht3)
    hns = (hn0, hn1, hn2, hn3)
    yns = (yn0, yn1, yn2, yn3)
    wc = wc_ref[...]
    wm2 = wm2_ref[...]
    bm2 = bm2_ref[...]
    wdec = wdec_ref[...]
    bdec = bdec_ref[...]
    for k in range(4):
        gt = gt_full[k * _H:(k + 1) * _H, :]
        yh = jnp.concatenate([yts[k][...], hts[k][...]], axis=0)
        m1 = gt + jnp.dot(wc, yh, preferred_element_type=jnp.float32)
        m1 = jnp.where(m1 > 0, m1, 0.01 * m1)
        m2 = jnp.dot(wm2, m1, preferred_element_type=jnp.float32) + bm2
        m2 = jnp.where(m2 > 0, m2, 0.01 * m2)
        hns[k][...] = m2
        lg = jnp.dot(wdec, m2, preferred_element_type=jnp.float32) + bdec
        mx = jnp.max(lg, axis=0, keepdims=True)
        ex = jnp.exp(lg - mx)
        yns[k][...] = ex / jnp.sum(ex, axis=0, keepdims=True)


def _edge_mlp(g, yt, ht, wc, wm2, bm2, wdec, bdec):
    grid = (_NBQ,)

    def edge_spec(rows, k):
        return pl.BlockSpec((rows, _BQ), lambda i, k=k: (0, k * _NBQ + i))

    return pl.pallas_call(
        _mlp_body,
        grid=grid,
        in_specs=[
            pl.BlockSpec((_BQ, 128), lambda i: (i, 0)),
            *[edge_spec(_MSG, k) for k in range(4)],
            *[edge_spec(_H, k) for k in range(4)],
            pl.BlockSpec((_H, _MSG + _H), lambda i: (0, 0)),
            pl.BlockSpec((_H, _H), lambda i: (0, 0)),
            pl.BlockSpec((_H, 1), lambda i: (0, 0)),
            pl.BlockSpec((_MSG, _H), lambda i: (0, 0)),
            pl.BlockSpec((_MSG, 1), lambda i: (0, 0)),
        ],
        out_specs=[
            *[pl.BlockSpec((_H, _BQ), lambda i: (0, i)) for _ in range(4)],
            *[pl.BlockSpec((_MSG, _BQ), lambda i: (0, i)) for _ in range(4)],
        ],
        out_shape=[
            *[jax.ShapeDtypeStruct((_H, _GQ), jnp.float32) for _ in range(4)],
            *[jax.ShapeDtypeStruct((_MSG, _GQ), jnp.float32) for _ in range(4)],
        ],
    )(g, *([yt] * 4), *([ht] * 4), wc, wm2, bm2, wdec, bdec)


def kernel(x, edge_index, h_msg, y_msg, W_in, b_in, W_enc, b_enc, W_m1, b_m1,
           W_m2, b_m2, W_u, b_u, W_dec, b_dec):
    w_m1a = W_m1[:, :_H]
    w_m1b = W_m1[:, _H:2 * _H]
    w_m1c = W_m1[:, 2 * _H:]
    wa = w_m1a @ W_in
    wb = w_m1b @ W_in
    wc = w_m1c @ W_enc
    ca = w_m1a @ b_in + w_m1c @ b_enc + b_m1
    cb = w_m1b @ b_in

    a, b = _node_tables(x, wa.T, wb.T, ca[None, :], cb[None, :])

    dst3 = edge_index[1].reshape(_IROWS, _SG)
    src3 = edge_index[0].reshape(_IROWS, _SG)
    g = _gather_add(a, b, dst3, src3)

    outs = _edge_mlp(
        g, y_msg.T, h_msg.T, wc, W_m2, b_m2[:, None], W_dec, b_dec[:, None]
    )
    hnewt = jnp.stack(outs[:4], axis=1).reshape(_H, _E)
    ynewt = jnp.stack(outs[4:], axis=1).reshape(_MSG, _E)
    return hnewt.T, ynewt.T


# speedup vs baseline: 7.0253x; 1.1603x over previous
import functools

import jax
import jax.numpy as jnp
from jax import lax
from jax.experimental import pallas as pl
from jax.experimental.pallas import tpu as pltpu
from jax.experimental.pallas import tpu_sc as plsc

_N = 100000
_E = 1600000
_H = 32
_MSG = 2

_NC = 2
_NS = 16
_NW = _NC * _NS
_S = 5
_ES = _E // _S
_EW = _ES // _NW
_SG = 125
_NSG = 10
_CH = _SG * _NSG
_CPW = _EW // _CH
_IROWS = _E // _SG
_IRW = _EW // _SG
_GQ = _ES // 4
_WPK = 8


def _node_tables_body(x_ref, wa_ref, wb_ref, ca_ref, cb_ref, a_ref, b_ref):
    xb = x_ref[...]
    a_ref[...] = (
        jnp.dot(xb, wa_ref[...], preferred_element_type=jnp.float32) + ca_ref[...]
    )
    b_ref[...] = (
        jnp.dot(xb, wb_ref[...], preferred_element_type=jnp.float32) + cb_ref[...]
    )


def _node_tables(x, wa_t, wb_t, ca, cb):
    bn = 10000
    grid = (_N // bn,)
    return pl.pallas_call(
        _node_tables_body,
        grid=grid,
        in_specs=[
            pl.BlockSpec((bn, 2), lambda i: (i, 0)),
            pl.BlockSpec((2, _H), lambda i: (0, 0)),
            pl.BlockSpec((2, _H), lambda i: (0, 0)),
            pl.BlockSpec((1, _H), lambda i: (0, 0)),
            pl.BlockSpec((1, _H), lambda i: (0, 0)),
        ],
        out_specs=[
            pl.BlockSpec((bn, _H), lambda i: (i, 0)),
            pl.BlockSpec((bn, _H), lambda i: (i, 0)),
        ],
        out_shape=[
            jax.ShapeDtypeStruct((_N, _H), jnp.float32),
            jax.ShapeDtypeStruct((_N, _H), jnp.float32),
        ],
    )(x, wa_t, wb_t, ca, cb)


def _gather_body(slab, a_hbm, b_hbm, dst_hbm, src_hbm, g_hbm, dstv, srcv, rows, sem_g):
    c = lax.axis_index("c")
    s = lax.axis_index("s")
    wid = s * _NC + c
    kcol = (wid // _WPK) * _H
    base_irow = slab * (_ES // _SG) + wid * _IRW
    base_grow = (wid % _WPK) * _EW

    def chunk_body(k, carry):
        irow0 = base_irow + k * _NSG
        grow0 = base_grow + k * _CH
        pltpu.sync_copy(dst_hbm.at[pl.ds(irow0, _NSG)], dstv)
        pltpu.sync_copy(src_hbm.at[pl.ds(irow0, _NSG)], srcv)

        def fire_a(j, cy):
            pltpu.async_copy(a_hbm.at[dstv.at[j]], rows.at[j], sem_g)
            return cy

        lax.fori_loop(0, _NSG, fire_a, 0)

        def drain_a(j, cy):
            pltpu.make_async_copy(a_hbm.at[dstv.at[j]], rows.at[j], sem_g).wait()
            return cy

        lax.fori_loop(0, _NSG, drain_a, 0)

        def fire_b(j, cy):
            pltpu.async_copy(b_hbm.at[srcv.at[j]], rows.at[j], sem_g, add=True)
            return cy

        lax.fori_loop(0, _NSG, fire_b, 0)

        def drain_b(j, cy):
            pltpu.make_async_copy(b_hbm.at[srcv.at[j]], rows.at[j], sem_g).wait()
            return cy

        lax.fori_loop(0, _NSG, drain_b, 0)

        def put_row(j, cy):
            pltpu.sync_copy(
                rows.at[j],
                g_hbm.at[pl.ds(grow0 + j * _SG, _SG), pl.ds(kcol, _H)],
            )
            return cy

        lax.fori_loop(0, _NSG, put_row, 0)
        return carry

    lax.fori_loop(0, _CPW, chunk_body, 0)


def _gather_add(a, b, dst3, src3, slab):
    mesh = plsc.VectorSubcoreMesh(
        core_axis_name="c", subcore_axis_name="s", num_cores=_NC, num_subcores=_NS
    )
    fn = pl.kernel(
        functools.partial(_gather_body, slab),
        out_type=jax.ShapeDtypeStruct((_GQ, 128), jnp.float32),
        mesh=mesh,
        scratch_types=[
            pltpu.VMEM((_NSG, _SG), jnp.int32),
            pltpu.VMEM((_NSG, _SG), jnp.int32),
            pltpu.VMEM((_NSG, _SG, _H), jnp.float32),
            pltpu.SemaphoreType.DMA,
        ],
        compiler_params=pltpu.CompilerParams(use_tc_tiling_on_sc=False),
    )
    return fn(a, b, dst3, src3)


_BQ = 3200
_NBQ = _GQ // _BQ


def _mlp_body(
    g_ref, yt0, yt1, yt2, yt3, ht0, ht1, ht2, ht3,
    wc_ref, wm2_ref, bm2_ref, wdec_ref, bdec_ref,
    hn0, hn1, hn2, hn3, yn0, yn1, yn2, yn3,
):
    gt_full = g_ref[...].T
    yts = (yt0, yt1, yt2, yt3)
    hts = (ht0, ht1, ht2, ht3)
    hns = (hn0, hn1, hn2, hn3)
    yns = (yn0, yn1, yn2, yn3)
    wc = wc_ref[...]
    wm2 = wm2_ref[...]
    bm2 = bm2_ref[...]
    wdec = wdec_ref[...]
    bdec = bdec_ref[...]
    for k in range(4):
        gt = gt_full[k * _H:(k + 1) * _H, :]
        yh = jnp.concatenate([yts[k][...], hts[k][...]], axis=0)
        m1 = gt + jnp.dot(wc, yh, preferred_element_type=jnp.float32)
        m1 = jnp.where(m1 > 0, m1, 0.01 * m1)
        m2 = jnp.dot(wm2, m1, preferred_element_type=jnp.float32) + bm2
        m2 = jnp.where(m2 > 0, m2, 0.01 * m2)
        hns[k][...] = m2
        lg = jnp.dot(wdec, m2, preferred_element_type=jnp.float32) + bdec
        mx = jnp.max(lg, axis=0, keepdims=True)
        ex = jnp.exp(lg - mx)
        yns[k][...] = ex / jnp.sum(ex, axis=0, keepdims=True)


def _edge_mlp(g, yt, ht, wc, wm2, bm2, wdec, bdec, slab):
    grid = (_NBQ,)
    sb = slab * (_ES // _BQ)

    def edge_spec(rows, k):
        return pl.BlockSpec((rows, _BQ), lambda i, k=k: (0, sb + k * _NBQ + i))

    return pl.pallas_call(
        _mlp_body,
        grid=grid,
        in_specs=[
            pl.BlockSpec((_BQ, 128), lambda i: (i, 0)),
            *[edge_spec(_MSG, k) for k in range(4)],
            *[edge_spec(_H, k) for k in range(4)],
            pl.BlockSpec((_H, _MSG + _H), lambda i: (0, 0)),
            pl.BlockSpec((_H, _H), lambda i: (0, 0)),
            pl.BlockSpec((_H, 1), lambda i: (0, 0)),
            pl.BlockSpec((_MSG, _H), lambda i: (0, 0)),
            pl.BlockSpec((_MSG, 1), lambda i: (0, 0)),
        ],
        out_specs=[
            *[pl.BlockSpec((_H, _BQ), lambda i: (0, i)) for _ in range(4)],
            *[pl.BlockSpec((_MSG, _BQ), lambda i: (0, i)) for _ in range(4)],
        ],
        out_shape=[
            *[jax.ShapeDtypeStruct((_H, _GQ), jnp.float32) for _ in range(4)],
            *[jax.ShapeDtypeStruct((_MSG, _GQ), jnp.float32) for _ in range(4)],
        ],
    )(g, *([yt] * 4), *([ht] * 4), wc, wm2, bm2, wdec, bdec)


def _mlp_and_pack(gs, yt, ht, wc, wm2, bm2, wdec, bdec):
    hparts, yparts = [], []
    for s in range(_S):
        outs = _edge_mlp(gs[s], yt, ht, wc, wm2, bm2, wdec, bdec, s)
        hparts.extend(outs[:4])
        yparts.extend(outs[4:])
    hnewt = jnp.stack(hparts, axis=1).reshape(_H, _E)
    ynewt = jnp.stack(yparts, axis=1).reshape(_MSG, _E)
    return hnewt, ynewt


def kernel(x, edge_index, h_msg, y_msg, W_in, b_in, W_enc, b_enc, W_m1, b_m1,
           W_m2, b_m2, W_u, b_u, W_dec, b_dec):
    w_m1a = W_m1[:, :_H]
    w_m1b = W_m1[:, _H:2 * _H]
    w_m1c = W_m1[:, 2 * _H:]
    wa = w_m1a @ W_in
    wb = w_m1b @ W_in
    wc = w_m1c @ W_enc
    ca = w_m1a @ b_in + w_m1c @ b_enc + b_m1
    cb = w_m1b @ b_in

    a, b = _node_tables(x, wa.T, wb.T, ca[None, :], cb[None, :])

    dst3 = edge_index[1].reshape(_IROWS, _SG)
    src3 = edge_index[0].reshape(_IROWS, _SG)
    gs = [_gather_add(a, b, dst3, src3, s) for s in range(_S)]

    hnewt, ynewt = _mlp_and_pack(
        gs, y_msg.T, h_msg.T, wc, W_m2, b_m2[:, None], W_dec, b_dec[:, None]
    )
    return hnewt.T, ynewt.T
